# Initial kernel scaffold; baseline (speedup 1.0000x reference)
import functools

import jax
import jax.numpy as jnp
from jax import lax
from jax.experimental import pallas as pl
from jax.experimental.pallas import tpu as pltpu
from jax.experimental.pallas import tpu_sc as plsc

N = 10000
D = 128
G = 64

NC = 2
NS = 16
NW = NC * NS
CH = 320
SEGP = 72
DUMMY = 64
JCH = 80
NJ = CH // JCH


def _seg_body(x_hbm, b_hbm, sum_out, cnt_out, xv, idxv, onesv, zsum, zcnt,
              ssum, scnt):
    c = lax.axis_index("c")
    s = lax.axis_index("s")
    wid = s * NC + c
    own = wid * CH
    base = jnp.minimum(own, N - CH)

    @pl.when(s == 0)
    def _init():
        def zrow(i, carry):
            for k in range(D // 16):
                zsum[i, pl.ds(k * 16, 16)] = jnp.zeros((16,), jnp.float32)
            zcnt[i, pl.ds(0, 16)] = jnp.zeros((16,), jnp.float32)
            return carry
        lax.fori_loop(0, SEGP, zrow, 0)
        pltpu.sync_copy(zsum, ssum)
        pltpu.sync_copy(zcnt, scnt)

    def orow(i, carry):
        onesv[i, pl.ds(0, 16)] = jnp.ones((16,), jnp.float32)
        return carry
    lax.fori_loop(0, JCH, orow, 0)

    pltpu.sync_copy(x_hbm.at[pl.ds(base, CH)], xv)
    for j in range(NJ):
        pltpu.sync_copy(b_hbm.at[pl.ds(base + j * JCH, JCH)], idxv.at[j])

    for j in range(NJ):
        for k in range(JCH // 16):
            v = idxv[j, pl.ds(k * 16, 16)]
            gi = base + j * JCH + k * 16 + lax.broadcasted_iota(
                jnp.int32, (16,), 0)
            idxv[j, pl.ds(k * 16, 16)] = jnp.where(gi >= own, v, DUMMY)

    plsc.subcore_barrier()

    for j in range(NJ):
        pltpu.sync_copy(xv.at[pl.ds(j * JCH, JCH)], ssum.at[idxv.at[j]],
                        add=True)
        pltpu.sync_copy(onesv, scnt.at[idxv.at[j]], add=True)

    plsc.subcore_barrier()

    @pl.when(s == 0)
    def _emit():
        pltpu.sync_copy(ssum, sum_out.at[c])
        pltpu.sync_copy(scnt, cnt_out.at[c])


@functools.partial(
    pl.kernel,
    mesh=plsc.VectorSubcoreMesh(core_axis_name="c", subcore_axis_name="s"),
    out_type=[
        jax.ShapeDtypeStruct((NC, SEGP, D), jnp.float32),
        jax.ShapeDtypeStruct((NC, SEGP, 16), jnp.float32),
    ],
    scratch_types=[
        pltpu.VMEM((CH, D), jnp.float32),
        pltpu.VMEM((NJ, JCH), jnp.int32),
        pltpu.VMEM((JCH, 16), jnp.float32),
        pltpu.VMEM((SEGP, D), jnp.float32),
        pltpu.VMEM((SEGP, 16), jnp.float32),
        pltpu.VMEM_SHARED((SEGP, D), jnp.float32),
        pltpu.VMEM_SHARED((SEGP, 16), jnp.float32),
    ],
)
def _seg_kernel(x_hbm, b_hbm, sum_out, cnt_out, xv, idxv, onesv, zsum, zcnt,
                ssum, scnt):
    _seg_body(x_hbm, b_hbm, sum_out, cnt_out, xv, idxv, onesv, zsum, zcnt,
              ssum, scnt)


def _mlp_body(sum_ref, cnt_ref, u_ref, w1_ref, b1_ref, w2_ref, b2_ref,
              w3_ref, b3_ref, o_ref):
    seg = sum_ref[0, 0:G, :] + sum_ref[1, 0:G, :]
    cnt = cnt_ref[0, 0:G, 0:1] + cnt_ref[1, 0:G, 0:1]
    mean = seg / jnp.maximum(cnt, 1.0)
    h = jnp.concatenate([u_ref[...], mean], axis=1)
    dn = (((1,), (1,)), ((), ()))
    h = jnp.maximum(
        lax.dot_general(h, w1_ref[...], dn, precision=lax.Precision.HIGHEST,
                        preferred_element_type=jnp.float32) + b1_ref[...], 0.0)
    h = jnp.maximum(
        lax.dot_general(h, w2_ref[...], dn, precision=lax.Precision.HIGHEST,
                        preferred_element_type=jnp.float32) + b2_ref[...], 0.0)
    o_ref[...] = lax.dot_general(
        h, w3_ref[...], dn, precision=lax.Precision.HIGHEST,
        preferred_element_type=jnp.float32) + b3_ref[...]


def kernel(x, edge_index, u, batch, W1, b1, W2, b2, W3, b3):
    del edge_index
    sums, cnts = _seg_kernel(x, batch)
    out = pl.pallas_call(
        _mlp_body,
        out_shape=jax.ShapeDtypeStruct((G, W3.shape[0]), jnp.float32),
    )(sums, cnts, u, W1, b1.reshape(1, -1), W2, b2.reshape(1, -1),
      W3, b3.reshape(1, -1))
    return out


# speedup vs baseline: 3.7999x; 3.7999x over previous
import functools

import jax
import jax.numpy as jnp
from jax import lax
from jax.experimental import pallas as pl
from jax.experimental.pallas import tpu as pltpu
from jax.experimental.pallas import tpu_sc as plsc

N = 10000
D = 128
G = 64

NC = 2
NS = 16
NW = NC * NS
CH = 320
SEGP = 72
DUMMY = 64
JCH = 80
NJ = CH // JCH


def _seg_body(x_hbm, b_hbm, sum_out, cnt_out, xv, idxv, onesv, zsum, zcnt,
              ssum, scnt):
    c = lax.axis_index("c")
    s = lax.axis_index("s")
    wid = s * NC + c
    own = wid * CH
    base = jnp.minimum(own, N - CH)

    @pl.when(s == 0)
    def _init():
        def zrow(i, carry):
            for k in range(D // 16):
                zsum[i, pl.ds(k * 16, 16)] = jnp.zeros((16,), jnp.float32)
                zcnt[i, pl.ds(k * 16, 16)] = jnp.zeros((16,), jnp.float32)
            return carry
        lax.fori_loop(0, SEGP, zrow, 0)
        pltpu.sync_copy(zsum, ssum)
        pltpu.sync_copy(zcnt, scnt)

    def orow(i, carry):
        for k in range(D // 16):
            onesv[i, pl.ds(k * 16, 16)] = jnp.ones((16,), jnp.float32)
        return carry
    lax.fori_loop(0, JCH, orow, 0)

    pltpu.sync_copy(x_hbm.at[pl.ds(base, CH)], xv)
    for j in range(NJ):
        pltpu.sync_copy(b_hbm.at[pl.ds(base + j * JCH, JCH)], idxv.at[j])

    for j in range(NJ):
        for k in range(JCH // 16):
            v = idxv[j, pl.ds(k * 16, 16)]
            gi = base + j * JCH + k * 16 + lax.broadcasted_iota(
                jnp.int32, (16,), 0)
            idxv[j, pl.ds(k * 16, 16)] = jnp.where(gi >= own, v, DUMMY)

    plsc.subcore_barrier()

    for j in range(NJ):
        pltpu.sync_copy(xv.at[pl.ds(j * JCH, JCH)], ssum.at[idxv.at[j]],
                        add=True)
        pltpu.sync_copy(onesv, scnt.at[idxv.at[j]], add=True)

    plsc.subcore_barrier()

    @pl.when(s == 0)
    def _emit():
        pltpu.sync_copy(ssum, sum_out.at[c])
        pltpu.sync_copy(scnt, cnt_out.at[c])


@functools.partial(
    pl.kernel,
    mesh=plsc.VectorSubcoreMesh(core_axis_name="c", subcore_axis_name="s"),
    out_type=[
        jax.ShapeDtypeStruct((NC, SEGP, D), jnp.float32),
        jax.ShapeDtypeStruct((NC, SEGP, D), jnp.float32),
    ],
    scratch_types=[
        pltpu.VMEM((CH, D), jnp.float32),
        pltpu.VMEM((NJ, JCH), jnp.int32),
        pltpu.VMEM((JCH, D), jnp.float32),
        pltpu.VMEM((SEGP, D), jnp.float32),
        pltpu.VMEM((SEGP, D), jnp.float32),
        pltpu.VMEM_SHARED((SEGP, D), jnp.float32),
        pltpu.VMEM_SHARED((SEGP, D), jnp.float32),
    ],
)
def _seg_kernel(x_hbm, b_hbm, sum_out, cnt_out, xv, idxv, onesv, zsum, zcnt,
                ssum, scnt):
    _seg_body(x_hbm, b_hbm, sum_out, cnt_out, xv, idxv, onesv, zsum, zcnt,
              ssum, scnt)


def _mlp_body(sum_ref, cnt_ref, u_ref, w1_ref, b1_ref, w2_ref, b2_ref,
              w3_ref, b3_ref, o_ref):
    seg = sum_ref[0, 0:G, :] + sum_ref[1, 0:G, :]
    cnt = cnt_ref[0, 0:G, 0:1] + cnt_ref[1, 0:G, 0:1]
    mean = seg / jnp.maximum(cnt, 1.0)
    h = jnp.concatenate([u_ref[...], mean], axis=1)
    dn = (((1,), (1,)), ((), ()))
    h = jnp.maximum(
        lax.dot_general(h, w1_ref[...], dn, precision=lax.Precision.HIGHEST,
                        preferred_element_type=jnp.float32) + b1_ref[...], 0.0)
    h = jnp.maximum(
        lax.dot_general(h, w2_ref[...], dn, precision=lax.Precision.HIGHEST,
                        preferred_element_type=jnp.float32) + b2_ref[...], 0.0)
    o_ref[...] = lax.dot_general(
        h, w3_ref[...], dn, precision=lax.Precision.HIGHEST,
        preferred_element_type=jnp.float32) + b3_ref[...]


def kernel(x, edge_index, u, batch, W1, b1, W2, b2, W3, b3):
    del edge_index
    sums, cnts = _seg_kernel(x, batch)
    out = pl.pallas_call(
        _mlp_body,
        out_shape=jax.ShapeDtypeStruct((G, W3.shape[0]), jnp.float32),
    )(sums, cnts, u, W1, b1.reshape(1, -1), W2, b2.reshape(1, -1),
      W3, b3.reshape(1, -1))
    return out


# speedup vs baseline: 4.2827x; 1.1270x over previous
import functools

import jax
import jax.numpy as jnp
from jax import lax
from jax.experimental import pallas as pl
from jax.experimental.pallas import tpu as pltpu
from jax.experimental.pallas import tpu_sc as plsc

N = 10000
D = 128
G = 64

NC = 2
NS = 16
NW = NC * NS
CH = 320
NPAD = NW * CH
SEGP = 72
DUMMY = 64
JCH = 80
NJ = CH // JCH
BROWS = NPAD // D


def _seg_body(x_hbm, b_hbm, zsum_hbm, sum_out, xv, idxv, ssum):
    c = lax.axis_index("c")
    s = lax.axis_index("s")
    wid = s * NC + c
    own = wid * CH
    base = jnp.minimum(own, N - CH)

    @pl.when(s == 0)
    def _init():
        pltpu.sync_copy(zsum_hbm, ssum)

    pltpu.sync_copy(x_hbm.at[pl.ds(base, CH)], xv)
    for j in range(NJ):
        pltpu.sync_copy(b_hbm.at[pl.ds(base + j * JCH, JCH)], idxv.at[j])

    @pl.when(base < own)
    def _fixup():
        for j in range(NJ):
            for k in range(JCH // 16):
                v = idxv[j, pl.ds(k * 16, 16)]
                gi = base + j * JCH + k * 16 + lax.broadcasted_iota(
                    jnp.int32, (16,), 0)
                idxv[j, pl.ds(k * 16, 16)] = jnp.where(gi >= own, v, DUMMY)

    plsc.subcore_barrier()

    for j in range(NJ):
        pltpu.sync_copy(xv.at[pl.ds(j * JCH, JCH)], ssum.at[idxv.at[j]],
                        add=True)

    plsc.subcore_barrier()

    @pl.when(s == 0)
    def _emit():
        pltpu.sync_copy(ssum, sum_out.at[c])


@functools.partial(
    pl.kernel,
    mesh=plsc.VectorSubcoreMesh(core_axis_name="c", subcore_axis_name="s"),
    out_type=jax.ShapeDtypeStruct((NC, SEGP, D), jnp.float32),
    scratch_types=[
        pltpu.VMEM((CH, D), jnp.float32),
        pltpu.VMEM((NJ, JCH), jnp.int32),
        pltpu.VMEM_SHARED((SEGP, D), jnp.float32),
    ],
)
def _seg_kernel(x_hbm, b_hbm, zsum_hbm, sum_out, xv, idxv, ssum):
    _seg_body(x_hbm, b_hbm, zsum_hbm, sum_out, xv, idxv, ssum)


def _mlp_body(sum_ref, b2d_ref, u_ref, w1_ref, b1_ref, w2_ref, b2_ref,
              w3_ref, b3_ref, o_ref):
    seg = sum_ref[0, 0:G, :] + sum_ref[1, 0:G, :]
    segs = lax.broadcasted_iota(jnp.int32, (G, D), 0)
    csum = jnp.zeros((G, D), jnp.float32)
    for r in range(BROWS):
        csum = csum + jnp.where(segs == b2d_ref[r:r + 1, :], 1.0, 0.0)
    cnt = jnp.sum(csum, axis=1, keepdims=True)
    mean = seg / jnp.maximum(cnt, 1.0)
    h = jnp.concatenate([u_ref[...], mean], axis=1)
    dn = (((1,), (1,)), ((), ()))
    h = jnp.maximum(
        lax.dot_general(h, w1_ref[...], dn, precision=lax.Precision.HIGHEST,
                        preferred_element_type=jnp.float32) + b1_ref[...], 0.0)
    h = jnp.maximum(
        lax.dot_general(h, w2_ref[...], dn, precision=lax.Precision.HIGHEST,
                        preferred_element_type=jnp.float32) + b2_ref[...], 0.0)
    o_ref[...] = lax.dot_general(
        h, w3_ref[...], dn, precision=lax.Precision.HIGHEST,
        preferred_element_type=jnp.float32) + b3_ref[...]


def kernel(x, edge_index, u, batch, W1, b1, W2, b2, W3, b3):
    del edge_index
    zsum_in = jnp.zeros((SEGP, D), jnp.float32)
    sums = _seg_kernel(x, batch, zsum_in)
    b2d = jnp.pad(batch, (0, NPAD - N), constant_values=G).reshape(BROWS, D)
    out = pl.pallas_call(
        _mlp_body,
        out_shape=jax.ShapeDtypeStruct((G, W3.shape[0]), jnp.float32),
    )(sums, b2d, u, W1, b1.reshape(1, -1), W2, b2.reshape(1, -1),
      W3, b3.reshape(1, -1))
    return out


# speedup vs baseline: 4.8005x; 1.1209x over previous
import functools

import jax
import jax.numpy as jnp
from jax import lax
from jax.experimental import pallas as pl
from jax.experimental.pallas import tpu as pltpu
from jax.experimental.pallas import tpu_sc as plsc

N = 10000
D = 128
G = 64

NC = 2
NS = 16
NW = NC * NS
CH = 320
NPAD = NW * CH
SEGP = 72
DUMMY = 64
JCH = 80
NJ = CH // JCH
BROWS = NPAD // D


def _seg_body(x_hbm, b_hbm, sum_out, xv, idxv, zbuf, ssum, sx, si, so):
    c = lax.axis_index("c")
    s = lax.axis_index("s")
    wid = s * NC + c
    own = wid * CH
    base = jnp.minimum(own, N - CH)

    hx = [pltpu.async_copy(x_hbm.at[pl.ds(base + j * JCH, JCH)],
                           xv.at[pl.ds(j * JCH, JCH)], sx)
          for j in range(NJ)]
    hi = [pltpu.async_copy(b_hbm.at[pl.ds(base + j * JCH, JCH)],
                           idxv.at[j], si)
          for j in range(NJ)]

    @pl.when(s == 0)
    def _init():
        def zrow(i, carry):
            for k in range(D // 16):
                zbuf[i, pl.ds(k * 16, 16)] = jnp.zeros((16,), jnp.float32)
            return carry
        lax.fori_loop(0, SEGP, zrow, 0)
        pltpu.sync_copy(zbuf, ssum)

    for h in hi:
        h.wait()

    @pl.when(base < own)
    def _fixup():
        for j in range(NJ):
            for k in range(JCH // 16):
                v = idxv[j, pl.ds(k * 16, 16)]
                gi = base + j * JCH + k * 16 + lax.broadcasted_iota(
                    jnp.int32, (16,), 0)
                idxv[j, pl.ds(k * 16, 16)] = jnp.where(gi >= own, v, DUMMY)

    plsc.subcore_barrier()

    hs = []
    for j in range(NJ):
        hx[j].wait()
        hs.append(pltpu.async_copy(xv.at[pl.ds(j * JCH, JCH)],
                                   ssum.at[idxv.at[j]], so, add=True))
    for h in hs:
        h.wait()

    plsc.subcore_barrier()

    @pl.when(s == 0)
    def _emit():
        pltpu.sync_copy(ssum, sum_out.at[c])


@functools.partial(
    pl.kernel,
    mesh=plsc.VectorSubcoreMesh(core_axis_name="c", subcore_axis_name="s"),
    out_type=jax.ShapeDtypeStruct((NC, SEGP, D), jnp.float32),
    scratch_types=[
        pltpu.VMEM((CH, D), jnp.float32),
        pltpu.VMEM((NJ, JCH), jnp.int32),
        pltpu.VMEM((SEGP, D), jnp.float32),
        pltpu.VMEM_SHARED((SEGP, D), jnp.float32),
        pltpu.SemaphoreType.DMA,
        pltpu.SemaphoreType.DMA,
        pltpu.SemaphoreType.DMA,
    ],
)
def _seg_kernel(x_hbm, b_hbm, sum_out, xv, idxv, zbuf, ssum, sx, si, so):
    _seg_body(x_hbm, b_hbm, sum_out, xv, idxv, zbuf, ssum, sx, si, so)


def _cnt_body(b2d_ref, cnt_ref):
    segs = lax.broadcasted_iota(jnp.int32, (G, D), 0)
    csum = jnp.zeros((G, D), jnp.float32)
    for r in range(BROWS):
        csum = csum + jnp.where(segs == b2d_ref[r:r + 1, :], 1.0, 0.0)
    cnt_ref[...] = jnp.sum(csum, axis=1, keepdims=True)


def _mlp_body(sum_ref, cnt_ref, u_ref, w1_ref, b1_ref, w2_ref, b2_ref,
              w3_ref, b3_ref, o_ref):
    seg = sum_ref[0, 0:G, :] + sum_ref[1, 0:G, :]
    mean = seg / jnp.maximum(cnt_ref[...], 1.0)
    h = jnp.concatenate([u_ref[...], mean], axis=1)
    dn = (((1,), (1,)), ((), ()))
    h = jnp.maximum(
        lax.dot_general(h, w1_ref[...], dn, precision=lax.Precision.HIGHEST,
                        preferred_element_type=jnp.float32) + b1_ref[...], 0.0)
    h = jnp.maximum(
        lax.dot_general(h, w2_ref[...], dn, precision=lax.Precision.HIGHEST,
                        preferred_element_type=jnp.float32) + b2_ref[...], 0.0)
    o_ref[...] = lax.dot_general(
        h, w3_ref[...], dn, precision=lax.Precision.HIGHEST,
        preferred_element_type=jnp.float32) + b3_ref[...]


def kernel(x, edge_index, u, batch, W1, b1, W2, b2, W3, b3):
    del edge_index
    sums = _seg_kernel(x, batch)
    b2d = jnp.pad(batch, (0, NPAD - N), constant_values=G).reshape(BROWS, D)
    cnt = pl.pallas_call(
        _cnt_body,
        out_shape=jax.ShapeDtypeStruct((G, 1), jnp.float32),
    )(b2d)
    out = pl.pallas_call(
        _mlp_body,
        out_shape=jax.ShapeDtypeStruct((G, W3.shape[0]), jnp.float32),
    )(sums, cnt, u, W1, b1.reshape(1, -1), W2, b2.reshape(1, -1),
      W3, b3.reshape(1, -1))
    return out


# speedup vs baseline: 4.9472x; 1.0305x over previous
import functools

import jax
import jax.numpy as jnp
from jax import lax
from jax.experimental import pallas as pl
from jax.experimental.pallas import tpu as pltpu
from jax.experimental.pallas import tpu_sc as plsc

N = 10000
D = 128
G = 64

NC = 2
NS = 16
NW = NC * NS
CH = 320
NPAD = NW * CH
SEGP = 72
DUMMY = 64
REPL = 2
SSROWS = REPL * SEGP
JCH = 80
NJ = CH // JCH
BROWS = NPAD // D


def _seg_body(x_hbm, b_hbm, sum_out, xv, idxv, zbuf, ssum, sx, si, so):
    c = lax.axis_index("c")
    s = lax.axis_index("s")
    wid = s * NC + c
    own = wid * CH
    base = jnp.minimum(own, N - CH)

    hx = [pltpu.async_copy(x_hbm.at[pl.ds(base + j * JCH, JCH)],
                           xv.at[pl.ds(j * JCH, JCH)], sx)
          for j in range(NJ)]
    hi = [pltpu.async_copy(b_hbm.at[pl.ds(base + j * JCH, JCH)],
                           idxv.at[j], si)
          for j in range(NJ)]

    @pl.when(s < REPL)
    def _init():
        def zrow(i, carry):
            for k in range(D // 16):
                zbuf[i, pl.ds(k * 16, 16)] = jnp.zeros((16,), jnp.float32)
            return carry
        lax.fori_loop(0, SEGP, zrow, 0)
        pltpu.sync_copy(zbuf, ssum.at[pl.ds(s * SEGP, SEGP)])

    for h in hi:
        h.wait()

    off = (s % REPL) * SEGP
    for j in range(NJ):
        for k in range(JCH // 16):
            v = idxv[j, pl.ds(k * 16, 16)]
            gi = base + j * JCH + k * 16 + lax.broadcasted_iota(
                jnp.int32, (16,), 0)
            idxv[j, pl.ds(k * 16, 16)] = off + jnp.where(gi >= own, v, DUMMY)

    plsc.subcore_barrier()

    hs = []
    for j in range(NJ):
        hx[j].wait()
        hs.append(pltpu.async_copy(xv.at[pl.ds(j * JCH, JCH)],
                                   ssum.at[idxv.at[j]], so, add=True))
    for h in hs:
        h.wait()

    plsc.subcore_barrier()

    @pl.when(s == 0)
    def _emit():
        pltpu.sync_copy(ssum, sum_out.at[c])


@functools.partial(
    pl.kernel,
    mesh=plsc.VectorSubcoreMesh(core_axis_name="c", subcore_axis_name="s"),
    out_type=jax.ShapeDtypeStruct((NC, SSROWS, D), jnp.float32),
    scratch_types=[
        pltpu.VMEM((CH, D), jnp.float32),
        pltpu.VMEM((NJ, JCH), jnp.int32),
        pltpu.VMEM((SEGP, D), jnp.float32),
        pltpu.VMEM_SHARED((SSROWS, D), jnp.float32),
        pltpu.SemaphoreType.DMA,
        pltpu.SemaphoreType.DMA,
        pltpu.SemaphoreType.DMA,
    ],
)
def _seg_kernel(x_hbm, b_hbm, sum_out, xv, idxv, zbuf, ssum, sx, si, so):
    _seg_body(x_hbm, b_hbm, sum_out, xv, idxv, zbuf, ssum, sx, si, so)


def _cnt_body(b2d_ref, cnt_ref):
    segs = lax.broadcasted_iota(jnp.int32, (G, D), 0)
    csum = jnp.zeros((G, D), jnp.float32)
    for r in range(BROWS):
        csum = csum + jnp.where(segs == b2d_ref[r:r + 1, :], 1.0, 0.0)
    cnt_ref[...] = jnp.sum(csum, axis=1, keepdims=True)


def _mlp_body(sum_ref, cnt_ref, u_ref, w1_ref, b1_ref, w2_ref, b2_ref,
              w3_ref, b3_ref, o_ref):
    seg = (sum_ref[0, 0:G, :] + sum_ref[0, SEGP:SEGP + G, :]
           + sum_ref[1, 0:G, :] + sum_ref[1, SEGP:SEGP + G, :])
    mean = seg / jnp.maximum(cnt_ref[...], 1.0)
    h = jnp.concatenate([u_ref[...], mean], axis=1)
    dn = (((1,), (1,)), ((), ()))
    h = jnp.maximum(
        lax.dot_general(h, w1_ref[...], dn, precision=lax.Precision.DEFAULT,
                        preferred_element_type=jnp.float32) + b1_ref[...], 0.0)
    h = jnp.maximum(
        lax.dot_general(h, w2_ref[...], dn, precision=lax.Precision.DEFAULT,
                        preferred_element_type=jnp.float32) + b2_ref[...], 0.0)
    o_ref[...] = lax.dot_general(
        h, w3_ref[...], dn, precision=lax.Precision.DEFAULT,
        preferred_element_type=jnp.float32) + b3_ref[...]


def kernel(x, edge_index, u, batch, W1, b1, W2, b2, W3, b3):
    del edge_index
    sums = _seg_kernel(x, batch)
    b2d = jnp.pad(batch, (0, NPAD - N), constant_values=G).reshape(BROWS, D)
    cnt = pl.pallas_call(
        _cnt_body,
        out_shape=jax.ShapeDtypeStruct((G, 1), jnp.float32),
    )(b2d)
    out = pl.pallas_call(
        _mlp_body,
        out_shape=jax.ShapeDtypeStruct((G, W3.shape[0]), jnp.float32),
    )(sums, cnt, u, W1, b1.reshape(1, -1), W2, b2.reshape(1, -1),
      W3, b3.reshape(1, -1))
    return out


# speedup vs baseline: 4.9513x; 1.0008x over previous
import functools

import jax
import jax.numpy as jnp
from jax import lax
from jax.experimental import pallas as pl
from jax.experimental.pallas import tpu as pltpu
from jax.experimental.pallas import tpu_sc as plsc

N = 10000
D = 128
G = 64

NC = 2
NS = 16
NW = NC * NS
CH = 320
NPAD = NW * CH
SEGP = 72
DUMMY = 64
REPL = 2
SSROWS = REPL * SEGP
JCH = 40
NJ = CH // JCH
BROWS = NPAD // D


def _seg_body(x_hbm, b_hbm, sum_out, xv, idxv, zbuf, ssum, sx, si, so):
    c = lax.axis_index("c")
    s = lax.axis_index("s")
    wid = s * NC + c
    own = wid * CH
    base = jnp.minimum(own, N - CH)

    hi = [pltpu.async_copy(b_hbm.at[pl.ds(base + j * JCH, JCH)],
                           idxv.at[j], si)
          for j in range(NJ)]
    hx = [pltpu.async_copy(x_hbm.at[pl.ds(base + j * JCH, JCH)],
                           xv.at[pl.ds(j * JCH, JCH)], sx)
          for j in range(NJ)]

    @pl.when(s < REPL)
    def _init():
        def zrow(i, carry):
            for k in range(D // 16):
                zbuf[i, pl.ds(k * 16, 16)] = jnp.zeros((16,), jnp.float32)
            return carry
        lax.fori_loop(0, SEGP, zrow, 0)
        pltpu.sync_copy(zbuf, ssum.at[pl.ds(s * SEGP, SEGP)])

    for h in hi:
        h.wait()

    off = (s % REPL) * SEGP
    for j in range(NJ):
        for k in range(JCH // 16):
            v = idxv[j, pl.ds(k * 16, 16)]
            gi = base + j * JCH + k * 16 + lax.broadcasted_iota(
                jnp.int32, (16,), 0)
            idxv[j, pl.ds(k * 16, 16)] = off + jnp.where(gi >= own, v, DUMMY)

    plsc.subcore_barrier()

    hs = []
    for j in range(NJ):
        hx[j].wait()
        hs.append(pltpu.async_copy(xv.at[pl.ds(j * JCH, JCH)],
                                   ssum.at[idxv.at[j]], so, add=True))
    for h in hs:
        h.wait()

    plsc.subcore_barrier()

    @pl.when(s == 0)
    def _emit():
        pltpu.sync_copy(ssum, sum_out.at[c])


@functools.partial(
    pl.kernel,
    mesh=plsc.VectorSubcoreMesh(core_axis_name="c", subcore_axis_name="s"),
    out_type=jax.ShapeDtypeStruct((NC, SSROWS, D), jnp.float32),
    scratch_types=[
        pltpu.VMEM((CH, D), jnp.float32),
        pltpu.VMEM((NJ, JCH), jnp.int32),
        pltpu.VMEM((SEGP, D), jnp.float32),
        pltpu.VMEM_SHARED((SSROWS, D), jnp.float32),
        pltpu.SemaphoreType.DMA,
        pltpu.SemaphoreType.DMA,
        pltpu.SemaphoreType.DMA,
    ],
)
def _seg_kernel(x_hbm, b_hbm, sum_out, xv, idxv, zbuf, ssum, sx, si, so):
    _seg_body(x_hbm, b_hbm, sum_out, xv, idxv, zbuf, ssum, sx, si, so)


def _cnt_body(b2d_ref, cnt_ref):
    segs = lax.broadcasted_iota(jnp.int32, (G, D), 0)
    csum = jnp.zeros((G, D), jnp.float32)
    for r in range(BROWS):
        csum = csum + jnp.where(segs == b2d_ref[r:r + 1, :], 1.0, 0.0)
    cnt_ref[...] = jnp.sum(csum, axis=1, keepdims=True)


def _mlp_body(sum_ref, cnt_ref, u_ref, w1_ref, b1_ref, w2_ref, b2_ref,
              w3_ref, b3_ref, o_ref):
    seg = (sum_ref[0, 0:G, :] + sum_ref[0, SEGP:SEGP + G, :]
           + sum_ref[1, 0:G, :] + sum_ref[1, SEGP:SEGP + G, :])
    mean = seg / jnp.maximum(cnt_ref[...], 1.0)
    h = jnp.concatenate([u_ref[...], mean], axis=1)
    dn = (((1,), (1,)), ((), ()))
    h = jnp.maximum(
        lax.dot_general(h, w1_ref[...], dn, precision=lax.Precision.DEFAULT,
                        preferred_element_type=jnp.float32) + b1_ref[...], 0.0)
    h = jnp.maximum(
        lax.dot_general(h, w2_ref[...], dn, precision=lax.Precision.DEFAULT,
                        preferred_element_type=jnp.float32) + b2_ref[...], 0.0)
    o_ref[...] = lax.dot_general(
        h, w3_ref[...], dn, precision=lax.Precision.DEFAULT,
        preferred_element_type=jnp.float32) + b3_ref[...]


def kernel(x, edge_index, u, batch, W1, b1, W2, b2, W3, b3):
    del edge_index
    sums = _seg_kernel(x, batch)
    b2d = jnp.pad(batch, (0, NPAD - N), constant_values=G).reshape(BROWS, D)
    cnt = pl.pallas_call(
        _cnt_body,
        out_shape=jax.ShapeDtypeStruct((G, 1), jnp.float32),
    )(b2d)
    out = pl.pallas_call(
        _mlp_body,
        out_shape=jax.ShapeDtypeStruct((G, W3.shape[0]), jnp.float32),
    )(sums, cnt, u, W1, b1.reshape(1, -1), W2, b2.reshape(1, -1),
      W3, b3.reshape(1, -1))
    return out


# speedup vs baseline: 4.9974x; 1.0093x over previous
import functools

import jax
import jax.numpy as jnp
from jax import lax
from jax.experimental import pallas as pl
from jax.experimental.pallas import tpu as pltpu
from jax.experimental.pallas import tpu_sc as plsc

N = 10000
D = 128
G = 64

NC = 2
NS = 16
NW = NC * NS
CH = 320
NPAD = NW * CH
SEGP = 72
DUMMY = 64
REPL = 2
SSROWS = REPL * SEGP
JCH = 80
NJ = CH // JCH
BROWS = NPAD // D


def _seg_body(x_hbm, b_hbm, sum_out, xv, idxv, zbuf, ssum, sx, si, so):
    c = lax.axis_index("c")
    s = lax.axis_index("s")
    wid = s * NC + c
    own = wid * CH
    base = jnp.minimum(own, N - CH)

    hi = [pltpu.async_copy(b_hbm.at[pl.ds(base + j * JCH, JCH)],
                           idxv.at[j], si)
          for j in range(NJ)]
    hx = [pltpu.async_copy(x_hbm.at[pl.ds(base + j * JCH, JCH)],
                           xv.at[pl.ds(j * JCH, JCH)], sx)
          for j in range(NJ)]

    @pl.when(s < REPL)
    def _init():
        def zrow(i, carry):
            for k in range(D // 16):
                zbuf[i, pl.ds(k * 16, 16)] = jnp.zeros((16,), jnp.float32)
            return carry
        lax.fori_loop(0, SEGP, zrow, 0)
        pltpu.sync_copy(zbuf, ssum.at[pl.ds(s * SEGP, SEGP)])

    for h in hi:
        h.wait()

    off = (s % REPL) * SEGP

    def fix(i, carry):
        j = i // (JCH // 16)
        k = i % (JCH // 16)
        v = idxv[j, pl.ds(k * 16, 16)]
        gi = base + j * JCH + k * 16 + lax.broadcasted_iota(
            jnp.int32, (16,), 0)
        idxv[j, pl.ds(k * 16, 16)] = off + jnp.where(gi >= own, v, DUMMY)
        return carry
    lax.fori_loop(0, NJ * (JCH // 16), fix, 0)

    plsc.subcore_barrier()

    hs = []
    for j in range(NJ):
        hx[j].wait()
        hs.append(pltpu.async_copy(xv.at[pl.ds(j * JCH, JCH)],
                                   ssum.at[idxv.at[j]], so, add=True))
    for h in hs:
        h.wait()

    plsc.subcore_barrier()

    @pl.when(s == 0)
    def _emit():
        pltpu.sync_copy(ssum, sum_out.at[c])


@functools.partial(
    pl.kernel,
    mesh=plsc.VectorSubcoreMesh(core_axis_name="c", subcore_axis_name="s"),
    out_type=jax.ShapeDtypeStruct((NC, SSROWS, D), jnp.float32),
    scratch_types=[
        pltpu.VMEM((CH, D), jnp.float32),
        pltpu.VMEM((NJ, JCH), jnp.int32),
        pltpu.VMEM((SEGP, D), jnp.float32),
        pltpu.VMEM_SHARED((SSROWS, D), jnp.float32),
        pltpu.SemaphoreType.DMA,
        pltpu.SemaphoreType.DMA,
        pltpu.SemaphoreType.DMA,
    ],
)
def _seg_kernel(x_hbm, b_hbm, sum_out, xv, idxv, zbuf, ssum, sx, si, so):
    _seg_body(x_hbm, b_hbm, sum_out, xv, idxv, zbuf, ssum, sx, si, so)


def _cnt_body(b2d_ref, cnt_ref):
    segs = lax.broadcasted_iota(jnp.int32, (G, D), 0)
    csum = jnp.zeros((G, D), jnp.float32)
    for r in range(BROWS):
        csum = csum + jnp.where(segs == b2d_ref[r:r + 1, :], 1.0, 0.0)
    cnt_ref[...] = jnp.sum(csum, axis=1, keepdims=True)


def _mlp_body(sum_ref, cnt_ref, u_ref, w1_ref, b1_ref, w2_ref, b2_ref,
              w3_ref, b3_ref, o_ref):
    seg = (sum_ref[0, 0:G, :] + sum_ref[0, SEGP:SEGP + G, :]
           + sum_ref[1, 0:G, :] + sum_ref[1, SEGP:SEGP + G, :])
    mean = seg / jnp.maximum(cnt_ref[...], 1.0)
    h = jnp.concatenate([u_ref[...], mean], axis=1)
    dn = (((1,), (1,)), ((), ()))
    h = jnp.maximum(
        lax.dot_general(h, w1_ref[...], dn, precision=lax.Precision.DEFAULT,
                        preferred_element_type=jnp.float32) + b1_ref[...], 0.0)
    h = jnp.maximum(
        lax.dot_general(h, w2_ref[...], dn, precision=lax.Precision.DEFAULT,
                        preferred_element_type=jnp.float32) + b2_ref[...], 0.0)
    o_ref[...] = lax.dot_general(
        h, w3_ref[...], dn, precision=lax.Precision.DEFAULT,
        preferred_element_type=jnp.float32) + b3_ref[...]


def kernel(x, edge_index, u, batch, W1, b1, W2, b2, W3, b3):
    del edge_index
    sums = _seg_kernel(x, batch)
    b2d = jnp.pad(batch, (0, NPAD - N), constant_values=G).reshape(BROWS, D)
    cnt = pl.pallas_call(
        _cnt_body,
        out_shape=jax.ShapeDtypeStruct((G, 1), jnp.float32),
    )(b2d)
    out = pl.pallas_call(
        _mlp_body,
        out_shape=jax.ShapeDtypeStruct((G, W3.shape[0]), jnp.float32),
    )(sums, cnt, u, W1, b1.reshape(1, -1), W2, b2.reshape(1, -1),
      W3, b3.reshape(1, -1))
    return out
